# Initial kernel scaffold; baseline (speedup 1.0000x reference)
import jax
import jax.numpy as jnp
from jax.experimental import pallas as pl


def kernel(x, V, U):
    raise NotImplementedError("write your pallas kernel here")


# speedup vs baseline: 4.3772x; 4.3772x over previous
import jax
import jax.numpy as jnp
from jax.experimental import pallas as pl
from jax.experimental.pallas import tpu as pltpu

D = 768
M = 64
B = 16
K = 2
EPS = 1e-8
LAMBDA = 1.0

TN = 256
MP = 128


def _fused_kernel(xf_ref, v_ref, ud_ref, ulast_ref,
                  xout_ref, stats_ref, acc_ref, sacc_ref):
    pid = pl.program_id(0)
    nprog = pl.num_programs(0)
    n_tok = nprog * TN

    v2 = v_ref[...]
    ud = ud_ref[...]
    ulast = ulast_ref[...]
    rv = 1.0 / jnp.maximum(jnp.sqrt(jnp.sum(v2 * v2, axis=0, keepdims=True)), EPS)
    ru = 1.0 / jnp.maximum(
        jnp.sqrt(jnp.sum(ud * ud, axis=0, keepdims=True) + ulast * ulast), EPS)
    vn = v2 * rv
    un = ud * ru

    xt = xf_ref[...]
    xn = xt / jnp.maximum(jnp.sqrt(jnp.sum(xt * xt, axis=1, keepdims=True)), EPS)

    h = jnp.dot(xn.astype(jnp.bfloat16), vn.astype(jnp.bfloat16),
                preferred_element_type=jnp.float32)

    jj = jax.lax.broadcasted_iota(jnp.int32, (M * B, MP), 0) // B
    mcol = jax.lax.broadcasted_iota(jnp.int32, (M * B, MP), 1)
    S = (jj == mcol).astype(jnp.float32)
    energy = jnp.dot(h * h, S, preferred_element_type=jnp.float32, precision=jax.lax.Precision.HIGHEST)

    midx = jax.lax.broadcasted_iota(jnp.int32, (TN, MP), 1)
    v1 = jnp.max(energy, axis=1, keepdims=True)
    idx1 = jnp.min(jnp.where(energy == v1, midx, MP + 1), axis=1, keepdims=True)
    oh1 = midx == idx1
    e2 = jnp.where(oh1, -1.0, energy)
    v2nd = jnp.max(e2, axis=1, keepdims=True)
    idx2 = jnp.min(jnp.where(e2 == v2nd, midx, MP + 1), axis=1, keepdims=True)
    maskM = (oh1 | (midx == idx2)).astype(jnp.float32)

    jj2 = jax.lax.broadcasted_iota(jnp.int32, (MP, M * B), 1) // B
    mrow = jax.lax.broadcasted_iota(jnp.int32, (MP, M * B), 0)
    S2 = (jj2 == mrow).astype(jnp.float32)
    mask_b = jnp.dot(maskM, S2, preferred_element_type=jnp.float32, precision=jax.lax.Precision.HIGHEST)
    hm = h * mask_b

    x_hat = jax.lax.dot_general(hm, vn, (((1,), (1,)), ((), ())),
                                preferred_element_type=jnp.float32, precision=jax.lax.Precision.HIGHEST)
    writes = jax.lax.dot_general(hm, un, (((1,), (1,)), ((), ())),
                                 preferred_element_type=jnp.float32, precision=jax.lax.Precision.HIGHEST)
    g = jnp.dot(writes, un, preferred_element_type=jnp.float32, precision=jax.lax.Precision.HIGHEST)

    resid = xn - x_hat
    xo = xn + LAMBDA * writes
    xout_ref[...] = xo / jnp.maximum(
        jnp.sqrt(jnp.sum(xo * xo, axis=1, keepdims=True)), EPS)

    @pl.when(pid == 0)
    def _():
        acc_ref[...] = jnp.zeros_like(acc_ref)
        for i in range(4):
            sacc_ref[i] = 0.0

    acc_ref[0:1, :] += jnp.sum(energy, axis=0, keepdims=True)
    acc_ref[1:2, :] += jnp.sum(maskM, axis=0, keepdims=True)
    diff = g - h
    sacc_ref[0] += jnp.sum(resid * resid)
    sacc_ref[1] += jnp.sum(x_hat * x_hat)
    sacc_ref[2] += jnp.sum(v1 + v2nd)
    sacc_ref[3] += jnp.sum(mask_b * diff * diff)

    @pl.when(pid == nprog - 1)
    def _():
        nf = jnp.float32(n_tok)
        uncaptured = sacc_ref[0] / nf
        recon = sacc_ref[1] / nf
        captured = sacc_ref[2] / nf
        writer = sacc_ref[3] / (nf * jnp.float32(K * B))
        lane = jax.lax.broadcasted_iota(jnp.int32, (1, MP), 1)
        valid = lane < M
        avg_e = acc_ref[0:1, :] / nf
        denom = jnp.maximum(jnp.sum(avg_e), EPS)
        probs = jnp.maximum(avg_e / denom, EPS)
        plogp = jnp.where(valid, probs * jnp.log(probs), 0.0)
        entropy = -jnp.sum(plogp) / jnp.log(jnp.float32(M))
        counts = acc_ref[1:2, :]
        expected = jnp.float32(K) / jnp.float32(M) * nf
        n_low = jnp.sum(jnp.where(valid & (counts <= 0.1 * expected), 1.0, 0.0))
        n_dead = jnp.sum(jnp.where(valid & (counts <= 0.01 * expected), 1.0, 0.0))
        stats_ref[0] = uncaptured + writer
        stats_ref[1] = uncaptured
        stats_ref[2] = entropy
        stats_ref[3] = captured
        stats_ref[4] = recon
        stats_ref[5] = n_low
        stats_ref[6] = n_dead


def kernel(x, V, U):
    n_tok = x.shape[0] * x.shape[1]
    grid = n_tok // TN
    xf = x.reshape(n_tok, D)
    v2 = V.reshape(D, M * B)
    u_t = jnp.transpose(U, (1, 0, 2)).reshape(D + 1, M * B)
    ud = u_t[:D]
    ulast = u_t[D:]

    x_out, stats = pl.pallas_call(
        _fused_kernel,
        grid=(grid,),
        in_specs=[
            pl.BlockSpec((TN, D), lambda i: (i, 0)),
            pl.BlockSpec((D, M * B), lambda i: (0, 0)),
            pl.BlockSpec((D, M * B), lambda i: (0, 0)),
            pl.BlockSpec((1, M * B), lambda i: (0, 0)),
        ],
        out_specs=[
            pl.BlockSpec((TN, D), lambda i: (i, 0)),
            pl.BlockSpec(memory_space=pltpu.SMEM),
        ],
        out_shape=[
            jax.ShapeDtypeStruct((n_tok, D), jnp.float32),
            jax.ShapeDtypeStruct((8,), jnp.float32),
        ],
        scratch_shapes=[
            pltpu.VMEM((8, MP), jnp.float32),
            pltpu.SMEM((8,), jnp.float32),
        ],
    )(xf, v2, ud, ulast)

    x_out = x_out.reshape(x.shape)
    return (x_out, stats[0], stats[1], stats[2], stats[3], stats[4],
            stats[5], stats[6])


# speedup vs baseline: 8.0078x; 1.8294x over previous
import jax
import jax.numpy as jnp
from jax.experimental import pallas as pl
from jax.experimental.pallas import tpu as pltpu

D = 768
M = 64
B = 16
K = 2
EPS = 1e-8
LAMBDA = 1.0

TN = 256
MP = 128


def _fused_kernel(xf_ref, v_ref, ud_ref, ulast_ref,
                  xout_ref, stats_ref, acc_ref, sacc_ref):
    pid = pl.program_id(0)
    nprog = pl.num_programs(0)
    n_tok = nprog * TN

    v2 = v_ref[...]
    ud = ud_ref[...]
    ulast = ulast_ref[...]
    rv = 1.0 / jnp.maximum(jnp.sqrt(jnp.sum(v2 * v2, axis=0, keepdims=True)), EPS)
    ru = 1.0 / jnp.maximum(
        jnp.sqrt(jnp.sum(ud * ud, axis=0, keepdims=True) + ulast * ulast), EPS)
    vn = v2 * rv
    un = ud * ru
    vnb = vn.astype(jnp.bfloat16)
    unb = un.astype(jnp.bfloat16)

    xt = xf_ref[...]
    xn = xt / jnp.maximum(jnp.sqrt(jnp.sum(xt * xt, axis=1, keepdims=True)), EPS)

    h = jnp.dot(xn.astype(jnp.bfloat16), vnb,
                preferred_element_type=jnp.float32)

    jj = jax.lax.broadcasted_iota(jnp.int32, (M * B, MP), 0) // B
    mcol = jax.lax.broadcasted_iota(jnp.int32, (M * B, MP), 1)
    S = (jj == mcol).astype(jnp.float32)
    energy = jnp.dot(h * h, S, preferred_element_type=jnp.float32, precision=jax.lax.Precision.HIGHEST)

    midx = jax.lax.broadcasted_iota(jnp.int32, (TN, MP), 1)
    v1 = jnp.max(energy, axis=1, keepdims=True)
    idx1 = jnp.min(jnp.where(energy == v1, midx, MP + 1), axis=1, keepdims=True)
    oh1 = midx == idx1
    e2 = jnp.where(oh1, -1.0, energy)
    v2nd = jnp.max(e2, axis=1, keepdims=True)
    idx2 = jnp.min(jnp.where(e2 == v2nd, midx, MP + 1), axis=1, keepdims=True)
    maskM = (oh1 | (midx == idx2)).astype(jnp.float32)

    jj2 = jax.lax.broadcasted_iota(jnp.int32, (MP, M * B), 1) // B
    mrow = jax.lax.broadcasted_iota(jnp.int32, (MP, M * B), 0)
    S2 = (jj2 == mrow).astype(jnp.bfloat16)
    mask_b = jnp.dot(maskM.astype(jnp.bfloat16), S2,
                     preferred_element_type=jnp.float32)
    hm = h * mask_b

    hmb = hm.astype(jnp.bfloat16)
    x_hat = jax.lax.dot_general(hmb, vnb, (((1,), (1,)), ((), ())),
                                preferred_element_type=jnp.float32)
    writes = jax.lax.dot_general(hmb, unb, (((1,), (1,)), ((), ())),
                                 preferred_element_type=jnp.float32)
    g = jnp.dot(writes.astype(jnp.bfloat16), unb,
                preferred_element_type=jnp.float32)

    resid = xn - x_hat
    xo = xn + LAMBDA * writes
    xout_ref[...] = xo / jnp.maximum(
        jnp.sqrt(jnp.sum(xo * xo, axis=1, keepdims=True)), EPS)

    @pl.when(pid == 0)
    def _():
        acc_ref[...] = jnp.zeros_like(acc_ref)
        for i in range(4):
            sacc_ref[i] = 0.0

    acc_ref[0:1, :] += jnp.sum(energy, axis=0, keepdims=True)
    acc_ref[1:2, :] += jnp.sum(maskM, axis=0, keepdims=True)
    diff = g - h
    sacc_ref[0] += jnp.sum(resid * resid)
    sacc_ref[1] += jnp.sum(x_hat * x_hat)
    sacc_ref[2] += jnp.sum(v1 + v2nd)
    sacc_ref[3] += jnp.sum(mask_b * diff * diff)

    @pl.when(pid == nprog - 1)
    def _():
        nf = jnp.float32(n_tok)
        uncaptured = sacc_ref[0] / nf
        recon = sacc_ref[1] / nf
        captured = sacc_ref[2] / nf
        writer = sacc_ref[3] / (nf * jnp.float32(K * B))
        lane = jax.lax.broadcasted_iota(jnp.int32, (1, MP), 1)
        valid = lane < M
        avg_e = acc_ref[0:1, :] / nf
        denom = jnp.maximum(jnp.sum(avg_e), EPS)
        probs = jnp.maximum(avg_e / denom, EPS)
        plogp = jnp.where(valid, probs * jnp.log(probs), 0.0)
        entropy = -jnp.sum(plogp) / jnp.log(jnp.float32(M))
        counts = acc_ref[1:2, :]
        expected = jnp.float32(K) / jnp.float32(M) * nf
        n_low = jnp.sum(jnp.where(valid & (counts <= 0.1 * expected), 1.0, 0.0))
        n_dead = jnp.sum(jnp.where(valid & (counts <= 0.01 * expected), 1.0, 0.0))
        stats_ref[0] = uncaptured + writer
        stats_ref[1] = uncaptured
        stats_ref[2] = entropy
        stats_ref[3] = captured
        stats_ref[4] = recon
        stats_ref[5] = n_low
        stats_ref[6] = n_dead


def kernel(x, V, U):
    n_tok = x.shape[0] * x.shape[1]
    grid = n_tok // TN
    xf = x.reshape(n_tok, D)
    v2 = V.reshape(D, M * B)
    u_t = jnp.transpose(U, (1, 0, 2)).reshape(D + 1, M * B)
    ud = u_t[:D]
    ulast = u_t[D:]

    x_out, stats = pl.pallas_call(
        _fused_kernel,
        grid=(grid,),
        in_specs=[
            pl.BlockSpec((TN, D), lambda i: (i, 0)),
            pl.BlockSpec((D, M * B), lambda i: (0, 0)),
            pl.BlockSpec((D, M * B), lambda i: (0, 0)),
            pl.BlockSpec((1, M * B), lambda i: (0, 0)),
        ],
        out_specs=[
            pl.BlockSpec((TN, D), lambda i: (i, 0)),
            pl.BlockSpec(memory_space=pltpu.SMEM),
        ],
        out_shape=[
            jax.ShapeDtypeStruct((n_tok, D), jnp.float32),
            jax.ShapeDtypeStruct((8,), jnp.float32),
        ],
        scratch_shapes=[
            pltpu.VMEM((8, MP), jnp.float32),
            pltpu.SMEM((8,), jnp.float32),
        ],
    )(xf, v2, ud, ulast)

    x_out = x_out.reshape(x.shape)
    return (x_out, stats[0], stats[1], stats[2], stats[3], stats[4],
            stats[5], stats[6])


# speedup vs baseline: 8.9791x; 1.1213x over previous
import jax
import jax.numpy as jnp
from jax.experimental import pallas as pl
from jax.experimental.pallas import tpu as pltpu

D = 768
M = 64
B = 16
K = 2
EPS = 1e-8
LAMBDA = 1.0

TN = 512
MP = 128


def _fused_kernel(xf_ref, v_ref, ud_ref, ulast_ref,
                  xout_ref, stats_ref, vnb_ref, unb_ref, acc_ref, sacc_ref):
    pid = pl.program_id(0)
    nprog = pl.num_programs(0)
    n_tok = nprog * TN

    @pl.when(pid == 0)
    def _():
        v2 = v_ref[...]
        ud = ud_ref[...]
        ulast = ulast_ref[...]
        rv = 1.0 / jnp.maximum(
            jnp.sqrt(jnp.sum(v2 * v2, axis=0, keepdims=True)), EPS)
        ru = 1.0 / jnp.maximum(
            jnp.sqrt(jnp.sum(ud * ud, axis=0, keepdims=True) + ulast * ulast),
            EPS)
        vnb_ref[...] = (v2 * rv).astype(jnp.bfloat16)
        unb_ref[...] = (ud * ru).astype(jnp.bfloat16)
        acc_ref[...] = jnp.zeros_like(acc_ref)
        for i in range(4):
            sacc_ref[i] = 0.0

    vnb = vnb_ref[...]
    unb = unb_ref[...]

    xt = xf_ref[...]
    xn = xt / jnp.maximum(jnp.sqrt(jnp.sum(xt * xt, axis=1, keepdims=True)), EPS)

    h = jnp.dot(xn.astype(jnp.bfloat16), vnb,
                preferred_element_type=jnp.float32)

    jj = jax.lax.broadcasted_iota(jnp.int32, (M * B, MP), 0) // B
    mcol = jax.lax.broadcasted_iota(jnp.int32, (M * B, MP), 1)
    S = (jj == mcol).astype(jnp.float32)
    energy = jnp.dot(h * h, S, preferred_element_type=jnp.float32,
                     precision=jax.lax.Precision.HIGHEST)

    midx = jax.lax.broadcasted_iota(jnp.int32, (TN, MP), 1)
    v1 = jnp.max(energy, axis=1, keepdims=True)
    idx1 = jnp.min(jnp.where(energy == v1, midx, MP + 1), axis=1, keepdims=True)
    oh1 = midx == idx1
    e2 = jnp.where(oh1, -1.0, energy)
    v2nd = jnp.max(e2, axis=1, keepdims=True)
    idx2 = jnp.min(jnp.where(e2 == v2nd, midx, MP + 1), axis=1, keepdims=True)
    maskM = (oh1 | (midx == idx2)).astype(jnp.float32)

    jj2 = jax.lax.broadcasted_iota(jnp.int32, (MP, M * B), 1) // B
    mrow = jax.lax.broadcasted_iota(jnp.int32, (MP, M * B), 0)
    S2 = (jj2 == mrow).astype(jnp.bfloat16)
    mask_b = jnp.dot(maskM.astype(jnp.bfloat16), S2,
                     preferred_element_type=jnp.float32)
    hm = h * mask_b

    hmb = hm.astype(jnp.bfloat16)
    x_hat = jax.lax.dot_general(hmb, vnb, (((1,), (1,)), ((), ())),
                                preferred_element_type=jnp.float32)
    writes = jax.lax.dot_general(hmb, unb, (((1,), (1,)), ((), ())),
                                 preferred_element_type=jnp.float32)
    g = jnp.dot(writes.astype(jnp.bfloat16), unb,
                preferred_element_type=jnp.float32)

    resid = xn - x_hat
    xo = xn + LAMBDA * writes
    xout_ref[...] = xo / jnp.maximum(
        jnp.sqrt(jnp.sum(xo * xo, axis=1, keepdims=True)), EPS)

    acc_ref[0:1, :] += jnp.sum(energy, axis=0, keepdims=True)
    acc_ref[1:2, :] += jnp.sum(maskM, axis=0, keepdims=True)
    diff = g - h
    sacc_ref[0] += jnp.sum(resid * resid)
    sacc_ref[1] += jnp.sum(x_hat * x_hat)
    sacc_ref[2] += jnp.sum(v1 + v2nd)
    sacc_ref[3] += jnp.sum(mask_b * diff * diff)

    @pl.when(pid == nprog - 1)
    def _():
        nf = jnp.float32(n_tok)
        uncaptured = sacc_ref[0] / nf
        recon = sacc_ref[1] / nf
        captured = sacc_ref[2] / nf
        writer = sacc_ref[3] / (nf * jnp.float32(K * B))
        lane = jax.lax.broadcasted_iota(jnp.int32, (1, MP), 1)
        valid = lane < M
        avg_e = acc_ref[0:1, :] / nf
        denom = jnp.maximum(jnp.sum(avg_e), EPS)
        probs = jnp.maximum(avg_e / denom, EPS)
        plogp = jnp.where(valid, probs * jnp.log(probs), 0.0)
        entropy = -jnp.sum(plogp) / jnp.log(jnp.float32(M))
        counts = acc_ref[1:2, :]
        expected = jnp.float32(K) / jnp.float32(M) * nf
        n_low = jnp.sum(jnp.where(valid & (counts <= 0.1 * expected), 1.0, 0.0))
        n_dead = jnp.sum(jnp.where(valid & (counts <= 0.01 * expected), 1.0, 0.0))
        stats_ref[0] = uncaptured + writer
        stats_ref[1] = uncaptured
        stats_ref[2] = entropy
        stats_ref[3] = captured
        stats_ref[4] = recon
        stats_ref[5] = n_low
        stats_ref[6] = n_dead


def kernel(x, V, U):
    n_tok = x.shape[0] * x.shape[1]
    grid = n_tok // TN
    xf = x.reshape(n_tok, D)
    v2 = V.reshape(D, M * B)
    u_t = jnp.transpose(U, (1, 0, 2)).reshape(D + 1, M * B)
    ud = u_t[:D]
    ulast = u_t[D:]

    x_out, stats = pl.pallas_call(
        _fused_kernel,
        grid=(grid,),
        in_specs=[
            pl.BlockSpec((TN, D), lambda i: (i, 0)),
            pl.BlockSpec((D, M * B), lambda i: (0, 0)),
            pl.BlockSpec((D, M * B), lambda i: (0, 0)),
            pl.BlockSpec((1, M * B), lambda i: (0, 0)),
        ],
        out_specs=[
            pl.BlockSpec((TN, D), lambda i: (i, 0)),
            pl.BlockSpec(memory_space=pltpu.SMEM),
        ],
        out_shape=[
            jax.ShapeDtypeStruct((n_tok, D), jnp.float32),
            jax.ShapeDtypeStruct((8,), jnp.float32),
        ],
        scratch_shapes=[
            pltpu.VMEM((D, M * B), jnp.bfloat16),
            pltpu.VMEM((D, M * B), jnp.bfloat16),
            pltpu.VMEM((8, MP), jnp.float32),
            pltpu.SMEM((8,), jnp.float32),
        ],
    )(xf, v2, ud, ulast)

    x_out = x_out.reshape(x.shape)
    return (x_out, stats[0], stats[1], stats[2], stats[3], stats[4],
            stats[5], stats[6])


# speedup vs baseline: 10.7758x; 1.2001x over previous
import jax
import jax.numpy as jnp
from jax.experimental import pallas as pl
from jax.experimental.pallas import tpu as pltpu

D = 768
M = 64
B = 16
K = 2
EPS = 1e-8
LAMBDA = 1.0

TN = 512
MP = 128


def _fused_kernel(xf_ref, v_ref, ud_ref, ulast_ref,
                  xout_ref, stats_ref, vnb_ref, unb_ref, acc_ref, sacc_ref):
    pid = pl.program_id(0)
    nprog = pl.num_programs(0)
    n_tok = nprog * TN

    @pl.when(pid == 0)
    def _():
        v2 = v_ref[...]
        ud = ud_ref[...]
        ulast = ulast_ref[...]
        rv = 1.0 / jnp.maximum(
            jnp.sqrt(jnp.sum(v2 * v2, axis=0, keepdims=True)), EPS)
        ru = 1.0 / jnp.maximum(
            jnp.sqrt(jnp.sum(ud * ud, axis=0, keepdims=True) + ulast * ulast),
            EPS)
        vnb_ref[...] = (v2 * rv).astype(jnp.bfloat16)
        unb_ref[...] = (ud * ru).astype(jnp.bfloat16)
        acc_ref[...] = jnp.zeros_like(acc_ref)
        for i in range(4):
            sacc_ref[i] = 0.0

    vnb = vnb_ref[...]
    unb = unb_ref[...]

    xt = xf_ref[...]
    xn = xt / jnp.maximum(jnp.sqrt(jnp.sum(xt * xt, axis=1, keepdims=True)), EPS)

    h = jnp.dot(xn.astype(jnp.bfloat16), vnb,
                preferred_element_type=jnp.float32)

    jj = jax.lax.broadcasted_iota(jnp.int32, (M * B, MP), 0) // B
    mcol = jax.lax.broadcasted_iota(jnp.int32, (M * B, MP), 1)
    S = (jj == mcol).astype(jnp.bfloat16)
    h2 = h * h
    h2a = h2.astype(jnp.bfloat16)
    r1 = h2 - h2a.astype(jnp.float32)
    h2b = r1.astype(jnp.bfloat16)
    h2c = (r1 - h2b.astype(jnp.float32)).astype(jnp.bfloat16)
    energy = (jnp.dot(h2a, S, preferred_element_type=jnp.float32)
              + jnp.dot(h2b, S, preferred_element_type=jnp.float32)
              + jnp.dot(h2c, S, preferred_element_type=jnp.float32))

    midx = jax.lax.broadcasted_iota(jnp.int32, (TN, MP), 1)
    v1 = jnp.max(energy, axis=1, keepdims=True)
    idx1 = jnp.min(jnp.where(energy == v1, midx, MP + 1), axis=1, keepdims=True)
    oh1 = midx == idx1
    e2 = jnp.where(oh1, -1.0, energy)
    v2nd = jnp.max(e2, axis=1, keepdims=True)
    idx2 = jnp.min(jnp.where(e2 == v2nd, midx, MP + 1), axis=1, keepdims=True)
    maskM = (oh1 | (midx == idx2)).astype(jnp.float32)

    jj2 = jax.lax.broadcasted_iota(jnp.int32, (MP, M * B), 1) // B
    mrow = jax.lax.broadcasted_iota(jnp.int32, (MP, M * B), 0)
    S2 = (jj2 == mrow).astype(jnp.bfloat16)
    mask_b = jnp.dot(maskM.astype(jnp.bfloat16), S2,
                     preferred_element_type=jnp.float32)
    hm = h * mask_b

    hmb = hm.astype(jnp.bfloat16)
    x_hat = jax.lax.dot_general(hmb, vnb, (((1,), (1,)), ((), ())),
                                preferred_element_type=jnp.float32)
    writes = jax.lax.dot_general(hmb, unb, (((1,), (1,)), ((), ())),
                                 preferred_element_type=jnp.float32)
    g = jnp.dot(writes.astype(jnp.bfloat16), unb,
                preferred_element_type=jnp.float32)

    resid = xn - x_hat
    xo = xn + LAMBDA * writes
    xout_ref[...] = xo / jnp.maximum(
        jnp.sqrt(jnp.sum(xo * xo, axis=1, keepdims=True)), EPS)

    acc_ref[0:1, :] += jnp.sum(energy, axis=0, keepdims=True)
    acc_ref[1:2, :] += jnp.sum(maskM, axis=0, keepdims=True)
    diff = g - h
    sacc_ref[0] += jnp.sum(resid * resid)
    sacc_ref[1] += jnp.sum(x_hat * x_hat)
    sacc_ref[2] += jnp.sum(v1 + v2nd)
    sacc_ref[3] += jnp.sum(mask_b * diff * diff)

    @pl.when(pid == nprog - 1)
    def _():
        nf = jnp.float32(n_tok)
        uncaptured = sacc_ref[0] / nf
        recon = sacc_ref[1] / nf
        captured = sacc_ref[2] / nf
        writer = sacc_ref[3] / (nf * jnp.float32(K * B))
        lane = jax.lax.broadcasted_iota(jnp.int32, (1, MP), 1)
        valid = lane < M
        avg_e = acc_ref[0:1, :] / nf
        denom = jnp.maximum(jnp.sum(avg_e), EPS)
        probs = jnp.maximum(avg_e / denom, EPS)
        plogp = jnp.where(valid, probs * jnp.log(probs), 0.0)
        entropy = -jnp.sum(plogp) / jnp.log(jnp.float32(M))
        counts = acc_ref[1:2, :]
        expected = jnp.float32(K) / jnp.float32(M) * nf
        n_low = jnp.sum(jnp.where(valid & (counts <= 0.1 * expected), 1.0, 0.0))
        n_dead = jnp.sum(jnp.where(valid & (counts <= 0.01 * expected), 1.0, 0.0))
        stats_ref[0] = uncaptured + writer
        stats_ref[1] = uncaptured
        stats_ref[2] = entropy
        stats_ref[3] = captured
        stats_ref[4] = recon
        stats_ref[5] = n_low
        stats_ref[6] = n_dead


def kernel(x, V, U):
    n_tok = x.shape[0] * x.shape[1]
    grid = n_tok // TN
    xf = x.reshape(n_tok, D)
    v2 = V.reshape(D, M * B)
    u_t = jnp.transpose(U, (1, 0, 2)).reshape(D + 1, M * B)
    ud = u_t[:D]
    ulast = u_t[D:]

    x_out, stats = pl.pallas_call(
        _fused_kernel,
        grid=(grid,),
        in_specs=[
            pl.BlockSpec((TN, D), lambda i: (i, 0)),
            pl.BlockSpec((D, M * B), lambda i: (0, 0)),
            pl.BlockSpec((D, M * B), lambda i: (0, 0)),
            pl.BlockSpec((1, M * B), lambda i: (0, 0)),
        ],
        out_specs=[
            pl.BlockSpec((TN, D), lambda i: (i, 0)),
            pl.BlockSpec(memory_space=pltpu.SMEM),
        ],
        out_shape=[
            jax.ShapeDtypeStruct((n_tok, D), jnp.float32),
            jax.ShapeDtypeStruct((8,), jnp.float32),
        ],
        scratch_shapes=[
            pltpu.VMEM((D, M * B), jnp.bfloat16),
            pltpu.VMEM((D, M * B), jnp.bfloat16),
            pltpu.VMEM((8, MP), jnp.float32),
            pltpu.SMEM((8,), jnp.float32),
        ],
    )(xf, v2, ud, ulast)

    x_out = x_out.reshape(x.shape)
    return (x_out, stats[0], stats[1], stats[2], stats[3], stats[4],
            stats[5], stats[6])


# speedup vs baseline: 11.0201x; 1.0227x over previous
import jax
import jax.numpy as jnp
from jax.experimental import pallas as pl
from jax.experimental.pallas import tpu as pltpu

D = 768
M = 64
B = 16
K = 2
EPS = 1e-8
LAMBDA = 1.0

TN = 1024
MP = 128


def _fused_kernel(xf_ref, v_ref, ud_ref, ulast_ref,
                  xout_ref, stats_ref, vnb_ref, unb_ref, acc_ref, sacc_ref):
    pid = pl.program_id(0)
    nprog = pl.num_programs(0)
    n_tok = nprog * TN

    @pl.when(pid == 0)
    def _():
        v2 = v_ref[...]
        ud = ud_ref[...]
        ulast = ulast_ref[...]
        rv = 1.0 / jnp.maximum(
            jnp.sqrt(jnp.sum(v2 * v2, axis=0, keepdims=True)), EPS)
        ru = 1.0 / jnp.maximum(
            jnp.sqrt(jnp.sum(ud * ud, axis=0, keepdims=True) + ulast * ulast),
            EPS)
        vnb_ref[...] = (v2 * rv).astype(jnp.bfloat16)
        unb_ref[...] = (ud * ru).astype(jnp.bfloat16)
        acc_ref[...] = jnp.zeros_like(acc_ref)
        for i in range(4):
            sacc_ref[i] = 0.0

    vnb = vnb_ref[...]
    unb = unb_ref[...]

    xt = xf_ref[...]
    xn = xt / jnp.maximum(jnp.sqrt(jnp.sum(xt * xt, axis=1, keepdims=True)), EPS)

    h = jnp.dot(xn.astype(jnp.bfloat16), vnb,
                preferred_element_type=jnp.float32)

    jj = jax.lax.broadcasted_iota(jnp.int32, (M * B, MP), 0) // B
    mcol = jax.lax.broadcasted_iota(jnp.int32, (M * B, MP), 1)
    S = (jj == mcol).astype(jnp.bfloat16)
    h2 = h * h
    h2a = h2.astype(jnp.bfloat16)
    r1 = h2 - h2a.astype(jnp.float32)
    h2b = r1.astype(jnp.bfloat16)
    h2c = (r1 - h2b.astype(jnp.float32)).astype(jnp.bfloat16)
    energy = (jnp.dot(h2a, S, preferred_element_type=jnp.float32)
              + jnp.dot(h2b, S, preferred_element_type=jnp.float32)
              + jnp.dot(h2c, S, preferred_element_type=jnp.float32))

    midx = jax.lax.broadcasted_iota(jnp.int32, (TN, MP), 1)
    v1 = jnp.max(energy, axis=1, keepdims=True)
    idx1 = jnp.min(jnp.where(energy == v1, midx, MP + 1), axis=1, keepdims=True)
    oh1 = midx == idx1
    e2 = jnp.where(oh1, -1.0, energy)
    v2nd = jnp.max(e2, axis=1, keepdims=True)
    idx2 = jnp.min(jnp.where(e2 == v2nd, midx, MP + 1), axis=1, keepdims=True)
    maskM = (oh1 | (midx == idx2)).astype(jnp.float32)

    jj2 = jax.lax.broadcasted_iota(jnp.int32, (MP, M * B), 1) // B
    mrow = jax.lax.broadcasted_iota(jnp.int32, (MP, M * B), 0)
    S2 = (jj2 == mrow).astype(jnp.bfloat16)
    mask_b = jnp.dot(maskM.astype(jnp.bfloat16), S2,
                     preferred_element_type=jnp.float32)
    hmb = h.astype(jnp.bfloat16) * mask_b.astype(jnp.bfloat16)
    x_hat = jax.lax.dot_general(hmb, vnb, (((1,), (1,)), ((), ())),
                                preferred_element_type=jnp.float32)
    writes = jax.lax.dot_general(hmb, unb, (((1,), (1,)), ((), ())),
                                 preferred_element_type=jnp.float32)
    g = jnp.dot(writes.astype(jnp.bfloat16), unb,
                preferred_element_type=jnp.float32)

    resid = xn - x_hat
    xo = xn + LAMBDA * writes
    xout_ref[...] = xo / jnp.maximum(
        jnp.sqrt(jnp.sum(xo * xo, axis=1, keepdims=True)), EPS)

    acc_ref[0:1, :] += jnp.sum(energy, axis=0, keepdims=True)
    acc_ref[1:2, :] += jnp.sum(maskM, axis=0, keepdims=True)
    diff = g - h
    sacc_ref[0] += jnp.sum(resid * resid)
    sacc_ref[1] += jnp.sum(x_hat * x_hat)
    sacc_ref[2] += jnp.sum(v1 + v2nd)
    sacc_ref[3] += jnp.sum(mask_b * diff * diff)

    @pl.when(pid == nprog - 1)
    def _():
        nf = jnp.float32(n_tok)
        uncaptured = sacc_ref[0] / nf
        recon = sacc_ref[1] / nf
        captured = sacc_ref[2] / nf
        writer = sacc_ref[3] / (nf * jnp.float32(K * B))
        lane = jax.lax.broadcasted_iota(jnp.int32, (1, MP), 1)
        valid = lane < M
        avg_e = acc_ref[0:1, :] / nf
        denom = jnp.maximum(jnp.sum(avg_e), EPS)
        probs = jnp.maximum(avg_e / denom, EPS)
        plogp = jnp.where(valid, probs * jnp.log(probs), 0.0)
        entropy = -jnp.sum(plogp) / jnp.log(jnp.float32(M))
        counts = acc_ref[1:2, :]
        expected = jnp.float32(K) / jnp.float32(M) * nf
        n_low = jnp.sum(jnp.where(valid & (counts <= 0.1 * expected), 1.0, 0.0))
        n_dead = jnp.sum(jnp.where(valid & (counts <= 0.01 * expected), 1.0, 0.0))
        stats_ref[0] = uncaptured + writer
        stats_ref[1] = uncaptured
        stats_ref[2] = entropy
        stats_ref[3] = captured
        stats_ref[4] = recon
        stats_ref[5] = n_low
        stats_ref[6] = n_dead


def kernel(x, V, U):
    n_tok = x.shape[0] * x.shape[1]
    grid = n_tok // TN
    xf = x.reshape(n_tok, D)
    v2 = V.reshape(D, M * B)
    u_t = jnp.transpose(U, (1, 0, 2)).reshape(D + 1, M * B)
    ud = u_t[:D]
    ulast = u_t[D:]

    x_out, stats = pl.pallas_call(
        _fused_kernel,
        grid=(grid,),
        in_specs=[
            pl.BlockSpec((TN, D), lambda i: (i, 0)),
            pl.BlockSpec((D, M * B), lambda i: (0, 0)),
            pl.BlockSpec((D, M * B), lambda i: (0, 0)),
            pl.BlockSpec((1, M * B), lambda i: (0, 0)),
        ],
        out_specs=[
            pl.BlockSpec((TN, D), lambda i: (i, 0)),
            pl.BlockSpec(memory_space=pltpu.SMEM),
        ],
        out_shape=[
            jax.ShapeDtypeStruct((n_tok, D), jnp.float32),
            jax.ShapeDtypeStruct((8,), jnp.float32),
        ],
        scratch_shapes=[
            pltpu.VMEM((D, M * B), jnp.bfloat16),
            pltpu.VMEM((D, M * B), jnp.bfloat16),
            pltpu.VMEM((8, MP), jnp.float32),
            pltpu.SMEM((8,), jnp.float32),
        ],
    )(xf, v2, ud, ulast)

    x_out = x_out.reshape(x.shape)
    return (x_out, stats[0], stats[1], stats[2], stats[3], stats[4],
            stats[5], stats[6])
